# baseline (speedup 1.0000x reference)
import functools

import jax
import jax.numpy as jnp
from jax import lax
from jax.experimental import pallas as pl
from jax.experimental.pallas import tpu as pltpu
from jax.experimental.pallas import tpu_sc as plsc

H = 768
E = 64
NSUB = 16
RPW = H // NSUB
NCH = H // 16
EPW = 8
L = 16

_f32 = jnp.float32
_i32 = jnp.int32


def _iota16():
    return lax.iota(_i32, L)


def _tree_sum(vs):
    vs = list(vs)
    while len(vs) > 1:
        vs = [vs[i] + vs[i + 1] for i in range(0, len(vs) - 1, 2)] + (
            [vs[-1]] if len(vs) % 2 else [])
    return vs[0]


def _sc_body(x_hbm, w1_hbm, b1_hbm, w2_hbm, b2_hbm, out_hbm,
             x_v, w1_v, b1_v, w2_v, b2_v, h_v, h_all, scr_v, lg_v,
             ll_v, o_v, shared_h, shared_l, sem, sem2):
    cid = lax.axis_index("c")
    sid = lax.axis_index("s")
    on0 = cid == 0
    s2 = jnp.logical_and(on0, sid < EPW)
    w0 = jnp.logical_and(on0, sid == 0)
    zeros = jnp.zeros((L,), _f32)

    @pl.when(on0)
    def _fire_dmas():
        base = sid * RPW
        pltpu.async_copy(x_hbm.at[0, 0], x_v, sem)
        pltpu.async_copy(w1_hbm.at[pl.ds(base, RPW)], w1_v, sem)
        pltpu.async_copy(b1_hbm.at[pl.ds(base, RPW)], b1_v, sem)

    @pl.when(s2)
    def _fire_w2():
        pltpu.async_copy(w2_hbm.at[pl.ds(sid * EPW, EPW)], w2_v, sem2)

    @pl.when(w0)
    def _fire_b2():
        pltpu.async_copy(b2_hbm, b2_v, sem2)

    @pl.when(on0)
    def _stage1():
        base = sid * RPW
        pltpu.make_async_copy(x_hbm.at[0, 0], x_v, sem).wait()
        pltpu.make_async_copy(w1_hbm.at[pl.ds(0, RPW)], w1_v, sem).wait()
        pltpu.make_async_copy(b1_hbm.at[pl.ds(0, RPW)], b1_v, sem).wait()

        def row_body(g, r, _):
            j = g * L + r
            jj = jnp.full((L,), j, _i32)
            accs = [zeros, zeros, zeros, zeros]
            for c in range(NCH):
                w = plsc.load_gather(w1_v, [jj, _iota16() + L * c])
                accs[c % 4] = accs[c % 4] + w * x_v[pl.ds(L * c, L)]
            acc = (accs[0] + accs[1]) + (accs[2] + accs[3])
            scr_v[pl.ds(r * L, L)] = acc
            return 0

        for g in range(3):
            lax.fori_loop(0, L, functools.partial(row_body, g), 0)
            cols = [plsc.load_gather(scr_v, [_iota16() * L + c])
                    for c in range(L)]
            hg = jnp.maximum(_tree_sum(cols) + b1_v[pl.ds(L * g, L)], 0.0)
            h_v[pl.ds(L * g, L)] = hg

        pltpu.sync_copy(h_v, shared_h.at[pl.ds(base, RPW)])

    plsc.subcore_barrier()

    @pl.when(s2)
    def _stage2():
        pltpu.make_async_copy(w2_hbm.at[pl.ds(0, EPW)], w2_v, sem2).wait()
        pltpu.sync_copy(shared_h, h_all)
        hc = [h_all[pl.ds(L * c, L)] for c in range(NCH)]
        for em in range(EPW):
            ee = jnp.full((L,), em, _i32)
            accs = [zeros, zeros, zeros, zeros]
            for c in range(NCH):
                wrow = plsc.load_gather(w2_v, [ee, _iota16() + L * c])
                accs[c % 4] = accs[c % 4] + wrow * hc[c]
            acc = (accs[0] + accs[1]) + (accs[2] + accs[3])
            scr_v[pl.ds(em * L, L)] = acc
        cols = [plsc.load_gather(scr_v, [_iota16() * L + c])
                for c in range(L)]
        lg_v[...] = _tree_sum(cols)
        pltpu.sync_copy(lg_v.at[pl.ds(0, EPW)],
                        shared_l.at[pl.ds(sid * EPW, EPW)])

    plsc.subcore_barrier()

    @pl.when(w0)
    def _finalize():
        pltpu.make_async_copy(b2_hbm, b2_v, sem2).wait()
        pltpu.sync_copy(shared_l, ll_v)
        lv = [ll_v[pl.ds(L * c, L)] + b2_v[pl.ds(L * c, L)] for c in range(4)]
        ids = [_iota16() + L * c for c in range(4)]
        big = jnp.full((L,), E, _i32)
        ninf = jnp.full((L,), -jnp.inf, _f32)

        m = jnp.maximum(jnp.maximum(lv[0], lv[1]), jnp.maximum(lv[2], lv[3]))
        v1 = jnp.max(m)
        v1b = jnp.full((L,), v1, _f32)
        i1cs = [jnp.min(jnp.where(lv[c] == v1b, ids[c], big)) for c in range(4)]
        i1 = jnp.minimum(jnp.minimum(i1cs[0], i1cs[1]),
                         jnp.minimum(i1cs[2], i1cs[3]))
        i1b = jnp.full((L,), i1, _i32)

        lv2 = [jnp.where(ids[c] == i1b, ninf, lv[c]) for c in range(4)]
        m2 = jnp.maximum(jnp.maximum(lv2[0], lv2[1]),
                         jnp.maximum(lv2[2], lv2[3]))
        v2 = jnp.max(m2)
        v2b = jnp.full((L,), v2, _f32)
        i2cs = [jnp.min(jnp.where(lv2[c] == v2b, ids[c], big)) for c in range(4)]
        i2 = jnp.minimum(jnp.minimum(i2cs[0], i2cs[1]),
                         jnp.minimum(i2cs[2], i2cs[3]))
        i2b = jnp.full((L,), i2, _i32)

        ev = jnp.exp(v2b - v1b)
        p1 = 1.0 / (1.0 + ev)
        p2 = ev / (1.0 + ev)
        zero = jnp.zeros((L,), _f32)
        for c in range(4):
            o = jnp.where(ids[c] == i1b, p1,
                          jnp.where(ids[c] == i2b, p2, zero))
            o_v[pl.ds(L * c, L)] = o
        pltpu.sync_copy(o_v, out_hbm)


def _make_sc_router():
    mesh = plsc.VectorSubcoreMesh(
        core_axis_name="c", subcore_axis_name="s",
        num_cores=2, num_subcores=NSUB)
    return pl.kernel(
        _sc_body,
        out_type=jax.ShapeDtypeStruct((E,), _f32),
        mesh=mesh,
        compiler_params=pltpu.CompilerParams(
            use_tc_tiling_on_sc=False, needs_layout_passes=False),
        scratch_types=[
            pltpu.VMEM((H,), _f32),
            pltpu.VMEM((RPW, H), _f32),
            pltpu.VMEM((RPW,), _f32),
            pltpu.VMEM((EPW, H), _f32),
            pltpu.VMEM((E,), _f32),
            pltpu.VMEM((RPW,), _f32),
            pltpu.VMEM((H,), _f32),
            pltpu.VMEM((256,), _f32),
            pltpu.VMEM((L,), _f32),
            pltpu.VMEM((E,), _f32),
            pltpu.VMEM((E,), _f32),
            pltpu.VMEM_SHARED((H,), _f32),
            pltpu.VMEM_SHARED((E,), _f32),
            pltpu.SemaphoreType.DMA,
            pltpu.SemaphoreType.DMA,
        ],
    )


_sc_router_cache = []


def kernel(x, W1, b1, W2, b2):
    if not _sc_router_cache:
        _sc_router_cache.append(_make_sc_router())
    return _sc_router_cache[0](x, W1, b1, W2, b2)


# speedup vs baseline: 1.0803x; 1.0803x over previous
import functools

import jax
import jax.numpy as jnp
from jax import lax
from jax.experimental import pallas as pl
from jax.experimental.pallas import tpu as pltpu
from jax.experimental.pallas import tpu_sc as plsc

H = 768
E = 64
NSUB = 16
RPW = H // NSUB
NCH = H // 16
EPW = 8
L = 16

_f32 = jnp.float32
_i32 = jnp.int32


def _iota16():
    return lax.iota(_i32, L)


def _tree_sum(vs):
    vs = list(vs)
    while len(vs) > 1:
        vs = [vs[i] + vs[i + 1] for i in range(0, len(vs) - 1, 2)] + (
            [vs[-1]] if len(vs) % 2 else [])
    return vs[0]


def _sc_body(x_hbm, w1_hbm, b1_hbm, w2_hbm, b2_hbm, out_hbm,
             x_v, w1_v, b1_v, w2_v, b2_v, h_v, h_all, scr_v, lg_v,
             ll_v, o_v, shared_h, shared_l, sem, sem2):
    cid = lax.axis_index("c")
    sid = lax.axis_index("s")
    on0 = cid == 0
    s2 = jnp.logical_and(on0, sid < EPW)
    w0 = jnp.logical_and(on0, sid == 0)
    zeros = jnp.zeros((L,), _f32)

    @pl.when(on0)
    def _fire_dmas():
        base = sid * RPW
        pltpu.async_copy(x_hbm.at[0, 0], x_v, sem)
        pltpu.async_copy(w1_hbm.at[pl.ds(base, RPW)], w1_v, sem)
        pltpu.async_copy(b1_hbm.at[pl.ds(base, RPW)], b1_v, sem)

    @pl.when(s2)
    def _fire_w2():
        pltpu.async_copy(w2_hbm.at[pl.ds(sid * EPW, EPW)], w2_v, sem2)

    @pl.when(w0)
    def _fire_b2():
        pltpu.async_copy(b2_hbm, b2_v, sem2)

    @pl.when(on0)
    def _stage1():
        base = sid * RPW
        pltpu.make_async_copy(x_hbm.at[0, 0], x_v, sem).wait()
        pltpu.make_async_copy(w1_hbm.at[pl.ds(0, RPW)], w1_v, sem).wait()
        pltpu.make_async_copy(b1_hbm.at[pl.ds(0, RPW)], b1_v, sem).wait()

        def row_body(g, r, _):
            j = g * L + r
            jj = jnp.full((L,), j, _i32)
            accs = [zeros, zeros, zeros, zeros]
            for c in range(NCH):
                w = plsc.load_gather(w1_v, [jj, _iota16() + L * c])
                accs[c % 4] = accs[c % 4] + w * x_v[pl.ds(L * c, L)]
            acc = (accs[0] + accs[1]) + (accs[2] + accs[3])
            scr_v[pl.ds(r * L, L)] = acc
            return 0

        for g in range(3):
            lax.fori_loop(0, L, functools.partial(row_body, g), 0)
            cols = [plsc.load_gather(scr_v, [_iota16() * L + c])
                    for c in range(L)]
            hg = jnp.maximum(_tree_sum(cols) + b1_v[pl.ds(L * g, L)], 0.0)
            h_v[pl.ds(L * g, L)] = hg

        pltpu.sync_copy(h_v, shared_h.at[pl.ds(base, RPW)])

    plsc.subcore_barrier()

    @pl.when(s2)
    def _stage2():
        pltpu.make_async_copy(w2_hbm.at[pl.ds(0, EPW)], w2_v, sem2).wait()
        pltpu.sync_copy(shared_h, h_all)
        hc = [h_all[pl.ds(L * c, L)] for c in range(NCH)]
        for em in range(EPW):
            ee = jnp.full((L,), em, _i32)
            accs = [zeros, zeros, zeros, zeros]
            for c in range(NCH):
                wrow = plsc.load_gather(w2_v, [ee, _iota16() + L * c])
                accs[c % 4] = accs[c % 4] + wrow * hc[c]
            acc = (accs[0] + accs[1]) + (accs[2] + accs[3])
            scr_v[pl.ds(em * L, L)] = acc
        cols = [plsc.load_gather(scr_v, [_iota16() * L + c])
                for c in range(L)]
        lg_v[...] = _tree_sum(cols)
        pltpu.sync_copy(lg_v.at[pl.ds(0, EPW)],
                        shared_l.at[pl.ds(sid * EPW, EPW)])

    plsc.subcore_barrier()

    @pl.when(w0)
    def _finalize():
        pltpu.make_async_copy(b2_hbm, b2_v, sem2).wait()
        pltpu.sync_copy(shared_l, ll_v)
        lv = [ll_v[pl.ds(L * c, L)] + b2_v[pl.ds(L * c, L)] for c in range(4)]
        ids = [_iota16() + L * c for c in range(4)]
        big = jnp.full((L,), E, _i32)
        ninf = jnp.full((L,), -jnp.inf, _f32)

        m = jnp.maximum(jnp.maximum(lv[0], lv[1]), jnp.maximum(lv[2], lv[3]))
        v1 = jnp.max(m)
        v1b = jnp.full((L,), v1, _f32)
        i1cs = [jnp.min(jnp.where(lv[c] == v1b, ids[c], big)) for c in range(4)]
        i1 = jnp.minimum(jnp.minimum(i1cs[0], i1cs[1]),
                         jnp.minimum(i1cs[2], i1cs[3]))
        i1b = jnp.full((L,), i1, _i32)

        lv2 = [jnp.where(ids[c] == i1b, ninf, lv[c]) for c in range(4)]
        m2 = jnp.maximum(jnp.maximum(lv2[0], lv2[1]),
                         jnp.maximum(lv2[2], lv2[3]))
        v2 = jnp.max(m2)
        v2b = jnp.full((L,), v2, _f32)
        i2cs = [jnp.min(jnp.where(lv2[c] == v2b, ids[c], big)) for c in range(4)]
        i2 = jnp.minimum(jnp.minimum(i2cs[0], i2cs[1]),
                         jnp.minimum(i2cs[2], i2cs[3]))
        i2b = jnp.full((L,), i2, _i32)

        ev = jnp.exp(v2b - v1b)
        p1 = 1.0 / (1.0 + ev)
        p2 = ev / (1.0 + ev)
        zero = jnp.zeros((L,), _f32)
        for c in range(4):
            o = jnp.where(ids[c] == i1b, p1,
                          jnp.where(ids[c] == i2b, p2, zero))
            o_v[pl.ds(L * c, L)] = o
        pltpu.sync_copy(o_v, out_hbm)


def _make_sc_router():
    mesh = plsc.VectorSubcoreMesh(
        core_axis_name="c", subcore_axis_name="s",
        num_cores=1, num_subcores=NSUB)
    return pl.kernel(
        _sc_body,
        out_type=jax.ShapeDtypeStruct((E,), _f32),
        mesh=mesh,
        compiler_params=pltpu.CompilerParams(
            use_tc_tiling_on_sc=False, needs_layout_passes=False),
        scratch_types=[
            pltpu.VMEM((H,), _f32),
            pltpu.VMEM((RPW, H), _f32),
            pltpu.VMEM((RPW,), _f32),
            pltpu.VMEM((EPW, H), _f32),
            pltpu.VMEM((E,), _f32),
            pltpu.VMEM((RPW,), _f32),
            pltpu.VMEM((H,), _f32),
            pltpu.VMEM((256,), _f32),
            pltpu.VMEM((L,), _f32),
            pltpu.VMEM((E,), _f32),
            pltpu.VMEM((E,), _f32),
            pltpu.VMEM_SHARED((H,), _f32),
            pltpu.VMEM_SHARED((E,), _f32),
            pltpu.SemaphoreType.DMA,
            pltpu.SemaphoreType.DMA,
        ],
    )


_sc_router_cache = []


def kernel(x, W1, b1, W2, b2):
    if not _sc_router_cache:
        _sc_router_cache.append(_make_sc_router())
    return _sc_router_cache[0](x, W1, b1, W2, b2)


# speedup vs baseline: 7.3144x; 6.7704x over previous
import jax
import jax.numpy as jnp
from jax.experimental import pallas as pl

_H = 768
_E = 64


def _router_body(x_ref, w1_ref, b1_ref, w2_ref, b2_ref, out_ref):
    x0 = x_ref[0]
    h = jax.lax.dot_general(
        x0, w1_ref[...], (((1,), (1,)), ((), ())),
        preferred_element_type=jnp.float32)
    h = jnp.maximum(h + b1_ref[...], 0.0)
    logits = jax.lax.dot_general(
        h, w2_ref[...], (((1,), (1,)), ((), ())),
        preferred_element_type=jnp.float32)
    logits = logits + b2_ref[...]

    ids = jax.lax.broadcasted_iota(jnp.int32, (1, _E), 1)
    v1 = jnp.max(logits, axis=1, keepdims=True)
    i1 = jnp.min(jnp.where(logits == v1, ids, _E), axis=1, keepdims=True)
    rest = jnp.where(ids == i1, -jnp.inf, logits)
    v2 = jnp.max(rest, axis=1, keepdims=True)
    i2 = jnp.min(jnp.where(rest == v2, ids, _E), axis=1, keepdims=True)

    e2 = jnp.exp(v2 - v1)
    denom = 1.0 + e2
    out_ref[...] = jnp.where(
        ids == i1, 1.0 / denom, jnp.where(ids == i2, e2 / denom, 0.0))


def kernel(x, W1, b1, W2, b2):
    out = pl.pallas_call(
        _router_body,
        grid=(1,),
        in_specs=[
            pl.BlockSpec((1, 1, _H), lambda i: (0, 0, 0)),
            pl.BlockSpec((_H, _H), lambda i: (0, 0)),
            pl.BlockSpec((1, _H), lambda i: (0, 0)),
            pl.BlockSpec((_E, _H), lambda i: (0, 0)),
            pl.BlockSpec((1, _E), lambda i: (0, 0)),
        ],
        out_specs=pl.BlockSpec((1, _E), lambda i: (0, 0)),
        out_shape=jax.ShapeDtypeStruct((1, _E), jnp.float32),
    )(x, W1, b1.reshape(1, _H), W2, b2.reshape(1, _E))
    return out.reshape(_E)
